# Initial kernel scaffold; baseline (speedup 1.0000x reference)
import jax
import jax.numpy as jnp
from jax.experimental import pallas as pl


def kernel(X, Edge_index, Edge_weight, Batching, conv1_w, conv1_att_src, conv1_att_dst, conv1_b, conv2_w, conv2_att_src, conv2_att_dst, conv2_b, chi_w1, chi_b1, chi_w2, chi_b2, chi_w3, chi_b3, rp_w1, rp_b1, rp_w2, rp_b2, rp_w3, rp_b3):
    raise NotImplementedError("write your pallas kernel here")


# speedup vs baseline: 34.7762x; 34.7762x over previous
import functools
import jax
import jax.numpy as jnp
from jax import lax
from jax.experimental import pallas as pl
from jax.experimental.pallas import tpu as pltpu, tpu_sc as plsc

N = 10000
E_REAL = 330000
NW = 32
CHUNK = 128
NCH = 81
T_PER_W = CHUNK * NCH
E_PAD = T_PER_W * NW
N_PAD = 10240
ROWS_PER_TILE = N_PAD // 16
F = 16
FW = 32


def _sc_edge_pass():
    mesh = plsc.VectorSubcoreMesh(core_axis_name="c", subcore_axis_name="s")

    @functools.partial(
        pl.kernel,
        mesh=mesh,
        out_type=jax.ShapeDtypeStruct((2 * N_PAD, FW), jnp.float32),
        compiler_params=pltpu.CompilerParams(needs_layout_passes=False, use_tc_tiling_on_sc=False),
        scratch_types=[
            pltpu.VMEM((N,), jnp.float32),
            pltpu.VMEM((N,), jnp.float32),
            pltpu.VMEM((CHUNK,), jnp.int32),
            pltpu.VMEM((CHUNK,), jnp.int32),
            pltpu.VMEM((CHUNK, F), jnp.float32),
            pltpu.VMEM((CHUNK, FW), jnp.float32),
            pltpu.VMEM((CHUNK,), jnp.float32),
            pltpu.VMEM_SHARED((N_PAD, FW), jnp.float32),
            pltpu.SemaphoreType.DMA,
        ],
    )
    def k(src_hbm, dst_hbm, asrc_hbm, adst_hbm, xp_hbm, zeros_hbm, out_hbm,
          asrc_v, adst_v, sidx_v, didx_v, rows_v, orows_v, ex_v, acc_sh, sem):
        cid = lax.axis_index("c")
        sid = lax.axis_index("s")
        wid = cid * 16 + sid
        base = wid * T_PER_W
        iota16 = lax.broadcasted_iota(jnp.int32, (16,), 0)

        row0 = sid * ROWS_PER_TILE
        pltpu.sync_copy(zeros_hbm.at[pl.ds(row0, ROWS_PER_TILE)],
                        acc_sh.at[pl.ds(row0, ROWS_PER_TILE)])
        pltpu.sync_copy(asrc_hbm, asrc_v)
        pltpu.sync_copy(adst_hbm, adst_v)
        plsc.subcore_barrier()

        def chunk_body(ch, _):
            off = base + ch * CHUNK
            pltpu.sync_copy(src_hbm.at[pl.ds(off, CHUNK)], sidx_v)
            pltpu.sync_copy(dst_hbm.at[pl.ds(off, CHUNK)], didx_v)
            gather = pltpu.async_copy(xp_hbm.at[sidx_v], rows_v, sem)

            def vec_body(j, _):
                s16 = sidx_v[pl.ds(j * 16, 16)]
                d16 = didx_v[pl.ds(j * 16, 16)]
                a = plsc.load_gather(asrc_v, [s16]) + plsc.load_gather(adst_v, [d16])
                a = jnp.where(a >= 0.0, a, 0.2 * a)
                ex = jnp.exp(a)
                gid = off + j * 16 + iota16
                ex = jnp.where(gid < E_REAL, ex, 0.0)
                ex_v[pl.ds(j * 16, 16)] = ex
                return 0

            lax.fori_loop(0, CHUNK // 16, vec_body, 0)
            gather.wait()

            def row_body(j, _):
                jj = jnp.full((16,), j, jnp.int32)
                exs = plsc.load_gather(ex_v, [jj])
                row = plsc.load_gather(rows_v, [jj, iota16])
                plsc.store_scatter(orows_v, [jj, iota16], row * exs)
                ex0 = jnp.where(iota16 == 0, exs, 0.0)
                plsc.store_scatter(orows_v, [jj, iota16 + F], ex0)
                return 0

            lax.fori_loop(0, CHUNK, row_body, 0)
            pltpu.sync_copy(orows_v, acc_sh.at[didx_v], add=True)
            return 0

        lax.fori_loop(0, NCH, chunk_body, 0)
        plsc.subcore_barrier()

        pltpu.sync_copy(acc_sh.at[pl.ds(row0, ROWS_PER_TILE)],
                        out_hbm.at[pl.ds(cid * N_PAD + row0, ROWS_PER_TILE)])

    return k


def _mm_t(a, w):
    return lax.dot_general(a, w, (((1,), (1,)), ((), ())),
                           preferred_element_type=jnp.float32)


def _tc1(x_ref, w_ref, aw_s_ref, aw_d_ref, xp_ref, as_ref, ad_ref):
    xp = _mm_t(x_ref[:], w_ref[:])
    xp_ref[:] = xp
    as_ref[:] = jnp.sum(xp * aw_s_ref[:], axis=1, keepdims=True)
    ad_ref[:] = jnp.sum(xp * aw_d_ref[:], axis=1, keepdims=True)


def _tc2(acc_ref, b_ref, w2_ref, aw_s_ref, aw_d_ref, xp_ref, as_ref, ad_ref):
    num = acc_ref[0:N, :] + acc_ref[N_PAD:N_PAD + N, :]
    h = num[:, 0:F] / (num[:, F:F + 1] + 1e-16) + b_ref[:]
    h = jnp.where(h >= 0.0, h, 0.01 * h)
    xp = _mm_t(h, w2_ref[:])
    xp_ref[:] = xp
    as_ref[:] = jnp.sum(xp * aw_s_ref[:], axis=1, keepdims=True)
    ad_ref[:] = jnp.sum(xp * aw_d_ref[:], axis=1, keepdims=True)


def _leaky(x):
    return jnp.where(x >= 0.0, x, 0.01 * x)


def _tc3(acc_ref, b_ref, batch_ref,
         cw1_ref, cb1_ref, cw2_ref, cb2_ref, cw3_ref, cb3_ref,
         rw1_ref, rb1_ref, rw2_ref, rb2_ref, rw3_ref, rb3_ref, out_ref):
    num = acc_ref[0:N, :] + acc_ref[N_PAD:N_PAD + N, :]
    out2 = _leaky(num[:, 0:F] / (num[:, F:F + 1] + 1e-16) + b_ref[:])

    gids = lax.broadcasted_iota(jnp.int32, (N, 64), 1)
    oneh = (batch_ref[:] == gids).astype(jnp.float32)
    seg = lax.dot_general(oneh, out2, (((0,), (0,)), ((), ())),
                          preferred_element_type=jnp.float32)
    counts = jnp.sum(oneh, axis=0, keepdims=True)
    mean = seg / jnp.maximum(counts, 1.0).T

    neg = jnp.float32(-jnp.inf)
    gcol = lax.broadcasted_iota(jnp.int32, (64, 1), 0)

    def g_body(g, maxp):
        m = batch_ref[:] == g
        mg = jnp.max(jnp.where(m, out2, neg), axis=0, keepdims=True)
        rowsel = gcol == g
        return jnp.where(rowsel, jnp.broadcast_to(mg, (64, F)), maxp)

    maxp = lax.fori_loop(0, 64, g_body, jnp.full((64, F), neg, jnp.float32))

    pooled = jnp.concatenate([mean, maxp], axis=1)

    chi = _leaky(_mm_t(pooled, cw1_ref[:]) + cb1_ref[:])
    chi = _leaky(_mm_t(chi, cw2_ref[:]) + cb2_ref[:])
    chi = jnp.sum(chi * cw3_ref[:], axis=1, keepdims=True) + cb3_ref[0, 0]
    rp = _leaky(_mm_t(pooled, rw1_ref[:]) + rb1_ref[:])
    rp = _leaky(_mm_t(rp, rw2_ref[:]) + rb2_ref[:])
    rp = jnp.sum(rp * rw3_ref[:], axis=1, keepdims=True) + rb3_ref[0, 0]
    out_ref[:] = jnp.concatenate([chi, rp], axis=1)


def _tc_call(fn, out_shapes, *args):
    return pl.pallas_call(fn, out_shape=out_shapes)(*args)


@jax.jit
def kernel(X, Edge_index, Edge_weight, Batching, conv1_w, conv1_att_src,
           conv1_att_dst, conv1_b, conv2_w, conv2_att_src, conv2_att_dst,
           conv2_b, chi_w1, chi_b1, chi_w2, chi_b2, chi_w3, chi_b3,
           rp_w1, rp_b1, rp_w2, rp_b2, rp_w3, rp_b3):
    f32 = jnp.float32
    loops = jnp.arange(N, dtype=jnp.int32)
    src = jnp.concatenate([Edge_index[0], loops])
    dst = jnp.concatenate([Edge_index[1], loops])
    src = jnp.pad(src, (0, E_PAD - E_REAL))
    dst = jnp.pad(dst, (0, E_PAD - E_REAL))
    zeros = jnp.zeros((N_PAD, FW), f32)

    sc_pass = _sc_edge_pass()

    xp1, as1, ad1 = _tc_call(
        _tc1,
        (jax.ShapeDtypeStruct((N, F), f32),
         jax.ShapeDtypeStruct((N, 1), f32),
         jax.ShapeDtypeStruct((N, 1), f32)),
        X, conv1_w, conv1_att_src.reshape(1, F), conv1_att_dst.reshape(1, F))
    acc1 = sc_pass(src, dst, as1.reshape(N), ad1.reshape(N), xp1, zeros)

    xp2, as2, ad2 = _tc_call(
        _tc2,
        (jax.ShapeDtypeStruct((N, F), f32),
         jax.ShapeDtypeStruct((N, 1), f32),
         jax.ShapeDtypeStruct((N, 1), f32)),
        acc1, conv1_b.reshape(1, F), conv2_w,
        conv2_att_src.reshape(1, F), conv2_att_dst.reshape(1, F))
    acc2 = sc_pass(src, dst, as2.reshape(N), ad2.reshape(N), xp2, zeros)

    out = _tc_call(
        _tc3,
        jax.ShapeDtypeStruct((64, 2), f32),
        acc2, conv2_b.reshape(1, F), Batching.reshape(N, 1),
        chi_w1, chi_b1.reshape(1, 32), chi_w2, chi_b2.reshape(1, 32),
        chi_w3, chi_b3.reshape(1, 1),
        rp_w1, rp_b1.reshape(1, 32), rp_w2, rp_b2.reshape(1, 32),
        rp_w3, rp_b3.reshape(1, 1))
    return out
